# baseline (speedup 1.0000x reference)
import functools

import jax
import jax.numpy as jnp
from jax.experimental import pallas as pl
from jax.experimental.pallas import tpu as pltpu

_V_CHUNK = 4096


def _lse_body(x_ref, a_ref, out_ref, m_ref, s_ref, g_ref, *, nv, v_total, chunk):
    i = pl.program_id(0)

    @pl.when(i == 0)
    def _init():
        m_ref[...] = jnp.full_like(m_ref[...], -jnp.inf)
        s_ref[...] = jnp.zeros_like(s_ref[...])
        g_ref[...] = jnp.zeros_like(g_ref[...])

    x = x_ref[...]
    col = jax.lax.broadcasted_iota(jnp.int32, x.shape, 1) + i * chunk
    x = jnp.where(col < v_total, x, -jnp.inf)

    a = a_ref[...]
    g_ref[...] += jnp.sum(jnp.where(col == a, x, 0.0), axis=1, keepdims=True)

    cm = jnp.max(x, axis=1, keepdims=True)
    m_old = m_ref[...]
    m_new = jnp.maximum(m_old, cm)
    s_chunk = jnp.sum(jnp.exp(x - cm), axis=1, keepdims=True)
    s_ref[...] = s_ref[...] * jnp.exp(m_old - m_new) + s_chunk * jnp.exp(cm - m_new)
    m_ref[...] = m_new

    @pl.when(i == nv - 1)
    def _done():
        out_ref[...] = g_ref[...] - (m_ref[...] + jnp.log(s_ref[...]))


def kernel(logits, actions):
    b, v = logits.shape
    a = actions.astype(jnp.int32)
    nv = pl.cdiv(v, _V_CHUNK)
    return pl.pallas_call(
        functools.partial(_lse_body, nv=nv, v_total=v, chunk=_V_CHUNK),
        grid=(nv,),
        in_specs=[
            pl.BlockSpec((b, _V_CHUNK), lambda i: (0, i)),
            pl.BlockSpec((b, 1), lambda i: (0, 0)),
        ],
        out_specs=pl.BlockSpec((b, 1), lambda i: (0, 0)),
        out_shape=jax.ShapeDtypeStruct((b, 1), jnp.float32),
        scratch_shapes=[
            pltpu.VMEM((b, 1), jnp.float32),
            pltpu.VMEM((b, 1), jnp.float32),
            pltpu.VMEM((b, 1), jnp.float32),
        ],
    )(logits, a)


# speedup vs baseline: 1.1624x; 1.1624x over previous
import functools

import jax
import jax.numpy as jnp
from jax.experimental import pallas as pl
from jax.experimental.pallas import tpu as pltpu

_V_CHUNK = 12544


def _lse_body(x_ref, a_ref, out_ref, m_ref, s_ref, g_ref, *, nv, v_total, chunk):
    i = pl.program_id(0)

    @pl.when(i == 0)
    def _init():
        m_ref[...] = jnp.full_like(m_ref[...], -jnp.inf)
        s_ref[...] = jnp.zeros_like(s_ref[...])
        g_ref[...] = jnp.zeros_like(g_ref[...])

    x = x_ref[...]
    col = jax.lax.broadcasted_iota(jnp.int32, x.shape, 1) + i * chunk
    x = jnp.where(col < v_total, x, -jnp.inf)

    a = a_ref[...]
    g_ref[...] += jnp.sum(jnp.where(col == a, x, 0.0), axis=1, keepdims=True)

    cm = jnp.max(x, axis=1, keepdims=True)
    m_old = m_ref[...]
    m_new = jnp.maximum(m_old, cm)
    s_chunk = jnp.sum(jnp.exp(x - cm), axis=1, keepdims=True)
    s_ref[...] = s_ref[...] * jnp.exp(m_old - m_new) + s_chunk * jnp.exp(cm - m_new)
    m_ref[...] = m_new

    @pl.when(i == nv - 1)
    def _done():
        out_ref[...] = g_ref[...] - (m_ref[...] + jnp.log(s_ref[...]))


def kernel(logits, actions):
    b, v = logits.shape
    a = actions.astype(jnp.int32)
    nv = pl.cdiv(v, _V_CHUNK)
    return pl.pallas_call(
        functools.partial(_lse_body, nv=nv, v_total=v, chunk=_V_CHUNK),
        grid=(nv,),
        in_specs=[
            pl.BlockSpec((b, _V_CHUNK), lambda i: (0, i)),
            pl.BlockSpec((b, 1), lambda i: (0, 0)),
        ],
        out_specs=pl.BlockSpec((b, 1), lambda i: (0, 0)),
        out_shape=jax.ShapeDtypeStruct((b, 1), jnp.float32),
        scratch_shapes=[
            pltpu.VMEM((b, 1), jnp.float32),
            pltpu.VMEM((b, 1), jnp.float32),
            pltpu.VMEM((b, 1), jnp.float32),
        ],
    )(logits, a)
